# baseline (speedup 1.0000x reference)
import functools

import jax
import jax.numpy as jnp
from jax import lax
from jax.experimental import pallas as pl
from jax.experimental.pallas import tpu as pltpu
from jax.experimental.pallas import tpu_sc as plsc

_HALF = 50048
_W1 = 48256
_RB = 64
_TAIL_VB = 4096


def _gather_fn(nc, ns, b_per_w, table_hbm, idx_hbm, out_hbm, idx_v, rows_v, sem):
    wid = lax.axis_index("s") * nc + lax.axis_index("c")
    base = wid * b_per_w
    pltpu.sync_copy(idx_hbm.at[pl.ds(base, b_per_w)], idx_v)
    pltpu.async_copy(table_hbm.at[idx_v], rows_v, sem).wait()
    pltpu.sync_copy(rows_v, out_hbm.at[pl.ds(base, b_per_w)])


def _sc_gather(table, idx_flat):
    info = plsc.get_sparse_core_info()
    nc, ns = info.num_cores, info.num_subcores
    nw = nc * ns
    b_total, d = idx_flat.shape[0], table.shape[1]
    b_per_w = b_total // nw
    mesh = plsc.VectorSubcoreMesh(core_axis_name="c", subcore_axis_name="s")
    kern = functools.partial(
        pl.kernel,
        mesh=mesh,
        out_type=jax.ShapeDtypeStruct((b_total, d), jnp.float32),
        scratch_types=[
            pltpu.VMEM((b_per_w,), jnp.int32),
            pltpu.VMEM((b_per_w, d), jnp.float32),
            pltpu.SemaphoreType.DMA,
        ],
        compiler_params=pltpu.CompilerParams(use_tc_tiling_on_sc=False),
    )(functools.partial(_gather_fn, nc, ns, b_per_w))
    return kern(table, idx_flat)


def _copy_half(acc, out_hbm, rows, j0, sem):
    if j0:
        return pltpu.make_async_copy(
            acc, out_hbm.at[pl.ds(rows, _RB), pl.ds(0, _HALF)], sem
        )
    return pltpu.make_async_copy(
        acc.at[:, pl.ds(0, _W1)],
        out_hbm.at[pl.ds(rows, _RB), pl.ds(_HALF, _W1)],
        sem,
    )


def _main_body(h_ref, w_ref, b_ref, out_hbm, a0, a1, s0, s1):
    j = pl.program_id(0)
    big_g = pl.program_id(1)
    accs, sems = (a0, a1), (s0, s1)

    for k in range(2):
        rows_now = (2 * big_g + k) * _RB

        @pl.when((big_g >= 1) & (j == 0))
        def _w0(k=k):
            _copy_half(accs[k], out_hbm, (2 * (big_g - 1) + k) * _RB, True, sems[k]).wait()

        @pl.when((big_g >= 1) & (j == 1))
        def _w1(k=k):
            _copy_half(accs[k], out_hbm, (2 * (big_g - 1) + k) * _RB, False, sems[k]).wait()

        @pl.when((j == 1) & (big_g == 0))
        def _w2(k=k):
            _copy_half(accs[k], out_hbm, (2 + k) * _RB, True, sems[k]).wait()

        accs[k][...] = (
            jnp.dot(
                h_ref[pl.ds(k * _RB, _RB), :],
                w_ref[...],
                preferred_element_type=jnp.float32,
            )
            + b_ref[...]
        )

        @pl.when(j == 0)
        def _s0(k=k, rows_now=rows_now):
            _copy_half(accs[k], out_hbm, rows_now, True, sems[k]).start()

        @pl.when(j == 1)
        def _s1(k=k, rows_now=rows_now):
            _copy_half(accs[k], out_hbm, rows_now, False, sems[k]).start()

    @pl.when((j == 1) & (big_g == 1))
    def _drain():
        for k in range(2):
            _copy_half(accs[k], out_hbm, (2 + k) * _RB, False, sems[k]).wait()


def _tail_body(h_ref, w_ref, b_ref, big_ref, o_ref):
    del big_ref
    o_ref[...] = (
        jnp.dot(h_ref[...], w_ref[...], preferred_element_type=jnp.float32)
        + b_ref[...]
    )


def _tc_project(hidden, proj_w, proj_b2d):
    r, h = hidden.shape
    v = proj_w.shape[1]
    main = pl.pallas_call(
        _main_body,
        grid=(2, 2),
        in_specs=[
            pl.BlockSpec((2 * _RB, h), lambda j, g: (g, 0)),
            pl.BlockSpec((h, _HALF), lambda j, g: (0, j)),
            pl.BlockSpec((1, _HALF), lambda j, g: (0, j)),
        ],
        out_specs=pl.BlockSpec(memory_space=pl.ANY),
        out_shape=jax.ShapeDtypeStruct((r, v), jnp.float32),
        scratch_shapes=[
            pltpu.VMEM((_RB, _HALF), jnp.float32),
            pltpu.VMEM((_RB, _HALF), jnp.float32),
            pltpu.SemaphoreType.DMA,
            pltpu.SemaphoreType.DMA,
        ],
    )(hidden, proj_w, proj_b2d)
    n_tiles = v // _TAIL_VB
    return pl.pallas_call(
        _tail_body,
        grid=(1,),
        in_specs=[
            pl.BlockSpec((r, h), lambda g: (0, 0)),
            pl.BlockSpec((h, _TAIL_VB), lambda g: (0, n_tiles)),
            pl.BlockSpec((1, _TAIL_VB), lambda g: (0, n_tiles)),
            pl.BlockSpec(memory_space=pl.ANY),
        ],
        out_specs=pl.BlockSpec((r, _TAIL_VB), lambda g: (0, n_tiles)),
        out_shape=jax.ShapeDtypeStruct((r, v), jnp.float32),
        input_output_aliases={3: 0},
    )(hidden, proj_w, proj_b2d, main)


def kernel(input_ids, emb_table, proj_w, proj_b):
    b, q = input_ids.shape
    v = proj_w.shape[1]
    idx_flat = input_ids.reshape(b * q).astype(jnp.int32)
    hidden = _sc_gather(emb_table, idx_flat)
    logits = _tc_project(hidden, proj_w, proj_b.reshape(1, v))
    return logits.reshape(b, q, v)


# speedup vs baseline: 1.0126x; 1.0126x over previous
import functools

import jax
import jax.numpy as jnp
from jax import lax
from jax.experimental import pallas as pl
from jax.experimental.pallas import tpu as pltpu
from jax.experimental.pallas import tpu_sc as plsc

_VB = 16384


def _gather_fn(nc, ns, b_per_w, table_hbm, idx_hbm, out_hbm, idx_v, rows_v, sem):
    wid = lax.axis_index("s") * nc + lax.axis_index("c")
    base = wid * b_per_w
    pltpu.sync_copy(idx_hbm.at[pl.ds(base, b_per_w)], idx_v)
    pltpu.async_copy(table_hbm.at[idx_v], rows_v, sem).wait()
    pltpu.sync_copy(rows_v, out_hbm.at[pl.ds(base, b_per_w)])


def _sc_gather(table, idx_flat):
    info = plsc.get_sparse_core_info()
    nc, ns = info.num_cores, info.num_subcores
    nw = nc * ns
    b_total, d = idx_flat.shape[0], table.shape[1]
    b_per_w = b_total // nw
    mesh = plsc.VectorSubcoreMesh(core_axis_name="c", subcore_axis_name="s")
    kern = functools.partial(
        pl.kernel,
        mesh=mesh,
        out_type=jax.ShapeDtypeStruct((b_total, d), jnp.float32),
        scratch_types=[
            pltpu.VMEM((b_per_w,), jnp.int32),
            pltpu.VMEM((b_per_w, d), jnp.float32),
            pltpu.SemaphoreType.DMA,
        ],
        compiler_params=pltpu.CompilerParams(use_tc_tiling_on_sc=False),
    )(functools.partial(_gather_fn, nc, ns, b_per_w))
    return kern(table, idx_flat)


def _proj_body(h_ref, w_ref, b_ref, o_ref):
    o_ref[...] = (
        jnp.dot(h_ref[...], w_ref[...], preferred_element_type=jnp.float32)
        + b_ref[...]
    )


def _tc_project(hidden, proj_w, proj_b2d):
    r, h = hidden.shape
    v = proj_w.shape[1]
    grid = (pl.cdiv(v, _VB),)
    return pl.pallas_call(
        _proj_body,
        grid=grid,
        in_specs=[
            pl.BlockSpec((r, h), lambda j: (0, 0)),
            pl.BlockSpec((h, _VB), lambda j: (0, j)),
            pl.BlockSpec((1, _VB), lambda j: (0, j)),
        ],
        out_specs=pl.BlockSpec((r, _VB), lambda j: (0, j)),
        out_shape=jax.ShapeDtypeStruct((r, v), jnp.float32),
    )(hidden, proj_w, proj_b2d)


def kernel(input_ids, emb_table, proj_w, proj_b):
    b, q = input_ids.shape
    v = proj_w.shape[1]
    idx_flat = input_ids.reshape(b * q).astype(jnp.int32)
    hidden = _sc_gather(emb_table, idx_flat)
    logits = _tc_project(hidden, proj_w, proj_b.reshape(1, v))
    return logits.reshape(b, q, v)
